# baseline (speedup 1.0000x reference)
import functools
import math

import jax
import jax.numpy as jnp
from jax import lax
from jax.experimental import pallas as pl
from jax.experimental.pallas import tpu as pltpu
from jax.experimental.pallas import tpu_sc as plsc

VOCAB = 100000
D = 300
SCALE = math.sqrt(300.0)
N = 4096 * 50
NC, NS = 2, 16
NW = NC * NS
PER_W = N // NW
C = 128
CHUNKS = PER_W // C

_mesh = plsc.VectorSubcoreMesh(core_axis_name="c", subcore_axis_name="s")


@functools.partial(
    pl.kernel,
    mesh=_mesh,
    out_type=jax.ShapeDtypeStruct((N, D), jnp.float32),
    compiler_params=pltpu.CompilerParams(use_tc_tiling_on_sc=False),
    scratch_types=[
        pltpu.VMEM((C,), jnp.int32),
        pltpu.VMEM((C, D), jnp.float32),
        pltpu.SemaphoreType.DMA,
    ],
)
def _emb_lookup(idx_hbm, table_hbm, out_hbm, idx_v, buf, sem):
    wid = lax.axis_index("s") * NC + lax.axis_index("c")
    base0 = wid * PER_W

    def chunk_body(k, carry):
        base = base0 + k * C
        pltpu.sync_copy(idx_hbm.at[pl.ds(base, C)], idx_v)
        pltpu.async_copy(table_hbm.at[idx_v], buf, sem).wait()

        pltpu.sync_copy(buf, out_hbm.at[pl.ds(base, C)])
        return carry

    lax.fori_loop(0, CHUNKS, chunk_body, 0)


def kernel(x, table):
    xf = x.reshape(-1).astype(jnp.int32)
    out = _emb_lookup(xf, table)
    return out.reshape(x.shape + (D,)) * jnp.float32(SCALE)


# speedup vs baseline: 1.2151x; 1.2151x over previous
import functools
import math

import jax
import jax.numpy as jnp
from jax import lax
from jax.experimental import pallas as pl
from jax.experimental.pallas import tpu as pltpu
from jax.experimental.pallas import tpu_sc as plsc

VOCAB = 100000
D = 300
DP = 304
SCALE = math.sqrt(300.0)
N = 4096 * 50
NC, NS = 2, 16
NW = NC * NS
PER_W = N // NW
C = 128
CHUNKS = PER_W // C

R_BLK = 2000
U_BLK = 4096


def _pad_scale_body(t_ref, o_ref):
    o_ref[:, :D] = t_ref[...] * SCALE
    o_ref[:, D:] = jnp.zeros((R_BLK, DP - D), jnp.float32)


_pad_scale = pl.pallas_call(
    _pad_scale_body,
    grid=(VOCAB // R_BLK,),
    in_specs=[pl.BlockSpec((R_BLK, D), lambda i: (i, 0))],
    out_specs=pl.BlockSpec((R_BLK, DP), lambda i: (i, 0)),
    out_shape=jax.ShapeDtypeStruct((VOCAB, DP), jnp.float32),
)


def _unpad_body(t_ref, o_ref):
    o_ref[...] = t_ref[:, :D]


_unpad = pl.pallas_call(
    _unpad_body,
    grid=(N // U_BLK,),
    in_specs=[pl.BlockSpec((U_BLK, DP), lambda i: (i, 0))],
    out_specs=pl.BlockSpec((U_BLK, D), lambda i: (i, 0)),
    out_shape=jax.ShapeDtypeStruct((N, D), jnp.float32),
)

_mesh = plsc.VectorSubcoreMesh(core_axis_name="c", subcore_axis_name="s")


@functools.partial(
    pl.kernel,
    mesh=_mesh,
    out_type=jax.ShapeDtypeStruct((N, DP), jnp.float32),
    compiler_params=pltpu.CompilerParams(use_tc_tiling_on_sc=False),
    scratch_types=[
        pltpu.VMEM((C,), jnp.int32),
        pltpu.VMEM((C, DP), jnp.float32),
        pltpu.SemaphoreType.DMA,
    ],
)
def _emb_lookup(idx_hbm, table_hbm, out_hbm, idx_v, buf, sem):
    wid = lax.axis_index("s") * NC + lax.axis_index("c")
    base0 = wid * PER_W

    def chunk_body(k, carry):
        base = base0 + k * C
        pltpu.sync_copy(idx_hbm.at[pl.ds(base, C)], idx_v)
        pltpu.async_copy(table_hbm.at[idx_v], buf, sem).wait()
        pltpu.sync_copy(buf, out_hbm.at[pl.ds(base, C)])
        return carry

    lax.fori_loop(0, CHUNKS, chunk_body, 0)


def kernel(x, table):
    xf = x.reshape(-1).astype(jnp.int32)
    out = _unpad(_emb_lookup(xf, _pad_scale(table)))
    return out.reshape(x.shape + (D,))
